# Initial kernel scaffold; baseline (speedup 1.0000x reference)
import jax
import jax.numpy as jnp
from jax.experimental import pallas as pl


def kernel(x, edge_index, W1, b1, W2, b2, gamma, beta, running_mean, running_var):
    raise NotImplementedError("write your pallas kernel here")


# speedup vs baseline: 7.1477x; 7.1477x over previous
import functools

import jax
import jax.numpy as jnp
from jax import lax
from jax.experimental import pallas as pl
from jax.experimental.pallas import tpu as pltpu
from jax.experimental.pallas import tpu_sc as plsc

N_NODES = 10000
D = 128
N_EDGES = 320000
BN_EPS = 1e-5

NC = 2
NS = 16
NW = NC * NS
EPW = N_EDGES // NW
CH = 80
NCH = EPW // CH
PAD_N = 10240
ROWS_PER_TILE = PAD_N // NS


def _sc_aggregate(x, src, dst, zeros):
  mesh = plsc.VectorSubcoreMesh(
      core_axis_name="c", subcore_axis_name="s", num_cores=NC, num_subcores=NS
  )

  @functools.partial(
      pl.kernel,
      out_type=jax.ShapeDtypeStruct((NC, PAD_N, D), jnp.float32),
      mesh=mesh,
      scratch_types=[
          pltpu.VMEM((NCH, CH), jnp.int32),
          pltpu.VMEM((NCH, CH), jnp.int32),
          pltpu.VMEM((CH, D), jnp.float32),
          pltpu.VMEM_SHARED((PAD_N, D), jnp.float32),
          pltpu.SemaphoreType.DMA,
      ],
  )
  def body(x_hbm, src_hbm, dst_hbm, zeros_hbm, out_hbm,
           src_v, dst_v, rows_v, agg_sh, sem):
    c = lax.axis_index("c")
    s = lax.axis_index("s")
    wid = s * NC + c

    pltpu.sync_copy(zeros_hbm.at[pl.ds(s * ROWS_PER_TILE, ROWS_PER_TILE)],
                    agg_sh.at[pl.ds(s * ROWS_PER_TILE, ROWS_PER_TILE)])
    pltpu.sync_copy(src_hbm.at[wid], src_v)
    pltpu.sync_copy(dst_hbm.at[wid], dst_v)
    plsc.subcore_barrier()

    def chunk(i, carry):
      pltpu.async_copy(x_hbm.at[src_v.at[i]], rows_v, sem).wait()
      pltpu.sync_copy(rows_v, agg_sh.at[dst_v.at[i]], add=True)
      return carry

    lax.fori_loop(0, NCH, chunk, 0)
    plsc.subcore_barrier()

    pltpu.sync_copy(agg_sh.at[pl.ds(s * ROWS_PER_TILE, ROWS_PER_TILE)],
                    out_hbm.at[c].at[pl.ds(s * ROWS_PER_TILE, ROWS_PER_TILE)])

  return body(x, src.reshape(NW, NCH, CH), dst.reshape(NW, NCH, CH), zeros)


BLK = 400


def _tc_body(x_ref, a0_ref, a1_ref, w1_ref, b1_ref, w2_ref, b2_ref, o_ref):
  xb = x_ref[...]
  h = xb + a0_ref[...] + a1_ref[...]
  h = jnp.maximum(
      jnp.dot(h, w1_ref[...], preferred_element_type=jnp.float32) + b1_ref[...],
      0.0)
  o_ref[...] = (
      jnp.dot(h, w2_ref[...], preferred_element_type=jnp.float32)
      + b2_ref[...] + xb)


def _tc_mlp(x, agg0, agg1, w1t, b1, w2f, b2f):
  grid = (N_NODES // BLK,)
  row_spec = pl.BlockSpec((BLK, D), lambda i: (i, 0))
  full_spec = pl.BlockSpec((D, D), lambda i: (0, 0))
  vec_spec = pl.BlockSpec((1, D), lambda i: (0, 0))
  return pl.pallas_call(
      _tc_body,
      grid=grid,
      in_specs=[row_spec, row_spec, row_spec,
                full_spec, vec_spec, full_spec, vec_spec],
      out_specs=row_spec,
      out_shape=jax.ShapeDtypeStruct((N_NODES, D), jnp.float32),
  )(x, agg0, agg1, w1t, b1.reshape(1, D), w2f, b2f.reshape(1, D))


def kernel(x, edge_index, W1, b1, W2, b2, gamma, beta, running_mean,
           running_var):
  src = edge_index[0].astype(jnp.int32)
  dst = edge_index[1].astype(jnp.int32)
  zeros = jnp.zeros((PAD_N, D), jnp.float32)
  agg = _sc_aggregate(x, src, dst, zeros)

  scale = gamma / jnp.sqrt(running_var + BN_EPS)
  w1t = W1.T
  w2f = W2.T * scale[None, :]
  b2f = b2 * scale + (beta - running_mean * scale)
  return _tc_mlp(x, agg[0, :N_NODES], agg[1, :N_NODES], w1t, b1, w2f, b2f)
